# baseline (speedup 1.0000x reference)
import functools

import jax
import jax.numpy as jnp
from jax import lax
from jax.experimental import pallas as pl
from jax.experimental.pallas import tpu as pltpu
from jax.experimental.pallas import tpu_sc as plsc

R, T = 4096, 200
D = 32
NC, NS = 2, 16
NW = NC * NS
NBUF = 8
NGROUPS = T // NBUF


_mesh = plsc.VectorSubcoreMesh(core_axis_name="c", subcore_axis_name="s")


@functools.partial(
    pl.kernel,
    out_type=jax.ShapeDtypeStruct((T, D, R), jnp.float32),
    mesh=_mesh,
    compiler_params=pltpu.CompilerParams(use_tc_tiling_on_sc=False),
    scratch_types=[
        pltpu.VMEM((NBUF, R), jnp.int32),
        pltpu.VMEM((NBUF, R), jnp.float32),
        pltpu.SemaphoreType.DMA((NBUF,)),
        pltpu.SemaphoreType.DMA((NBUF,)),
        pltpu.SemaphoreType.DMA((NBUF,)),
    ],
)
def _embed_sc(idx_hbm, table_hbm, out_hbm, idx_v, val_v, isem, gsem, ssem):
    d = lax.axis_index("s") * NC + lax.axis_index("c")
    row = table_hbm.at[d]

    def stage_idx(j, b):
        pltpu.async_copy(idx_hbm.at[j], idx_v.at[b], isem.at[b])

    def gather(b):
        pltpu.async_copy(row.at[idx_v.at[b]], val_v.at[b], gsem.at[b])

    def store(j, b):
        pltpu.async_copy(val_v.at[b], out_hbm.at[j, d], ssem.at[b])

    for b in range(NBUF):
        stage_idx(b, b)

    def group(g, carry):
        base = g * NBUF
        for b in range(NBUF):
            pltpu.make_async_copy(idx_hbm.at[base + b], idx_v.at[b], isem.at[b]).wait()
            gather(b)
        for b in range(NBUF):
            pltpu.make_async_copy(row.at[idx_v.at[b]], val_v.at[b], gsem.at[b]).wait()
            store(base + b, b)
        for b in range(NBUF):
            pltpu.make_async_copy(val_v.at[b], out_hbm.at[base + b, d], ssem.at[b]).wait()

            @pl.when(g + 1 < NGROUPS)
            def _():
                stage_idx(base + NBUF + b, b)

        return carry

    lax.fori_loop(0, NGROUPS, group, 0)


def kernel(token_ids, weight):
    out_p = _embed_sc(token_ids.T, weight.T)
    return jnp.transpose(out_p, (2, 0, 1))


# speedup vs baseline: 3.6881x; 3.6881x over previous
import functools

import jax
import jax.numpy as jnp
from jax import lax
from jax.experimental import pallas as pl
from jax.experimental.pallas import tpu as pltpu
from jax.experimental.pallas import tpu_sc as plsc

R, T = 4096, 200
D = 32
NC, NS = 2, 16
NW = NC * NS
RPW = R // NW
NBUF = 8
NGROUPS = RPW // NBUF


_mesh = plsc.VectorSubcoreMesh(core_axis_name="c", subcore_axis_name="s")


@functools.partial(
    pl.kernel,
    out_type=jax.ShapeDtypeStruct((R, T, D), jnp.float32),
    mesh=_mesh,
    compiler_params=pltpu.CompilerParams(use_tc_tiling_on_sc=False),
    scratch_types=[
        pltpu.VMEM((RPW, T), jnp.int32),
        pltpu.VMEM((NBUF, T, D), jnp.float32),
        pltpu.SemaphoreType.DMA((NBUF,)),
        pltpu.SemaphoreType.DMA((NBUF,)),
    ],
)
def _embed_sc(idx_hbm, table_hbm, out_hbm, idx_v, rows_v, gsem, ssem):
    wid = lax.axis_index("s") * NC + lax.axis_index("c")
    base = wid * RPW
    pltpu.sync_copy(idx_hbm.at[pl.ds(base, RPW)], idx_v)

    def gather(i, b):
        pltpu.async_copy(table_hbm.at[idx_v.at[i]], rows_v.at[b], gsem.at[b])

    for b in range(NBUF):
        gather(b, b)

    def group(g, carry):
        gbase = g * NBUF
        for b in range(NBUF):
            pltpu.make_async_copy(
                table_hbm.at[idx_v.at[gbase + b]], rows_v.at[b], gsem.at[b]
            ).wait()
            pltpu.async_copy(
                rows_v.at[b], out_hbm.at[base + gbase + b], ssem.at[b]
            )
        for b in range(NBUF):
            pltpu.make_async_copy(
                rows_v.at[b], out_hbm.at[base + gbase + b], ssem.at[b]
            ).wait()

            @pl.when(g + 1 < NGROUPS)
            def _():
                gather(gbase + NBUF + b, b)

        return carry

    lax.fori_loop(0, NGROUPS, group, 0)


def kernel(token_ids, weight):
    return _embed_sc(token_ids, weight)
